# baseline (speedup 1.0000x reference)
import functools

import jax
import jax.numpy as jnp
from jax import lax
from jax.experimental import pallas as pl
from jax.experimental.pallas import tpu as pltpu
from jax.experimental.pallas import tpu_sc as plsc

VOCAB = 1000000
EMBED = 64
B = 4096
L = 200
EPS = 1e-5

BL = B * L
NC, NS = 2, 16
NW = NC * NS
ROWS_PER_W = BL // NW
CHUNK = 800
NCHUNK = ROWS_PER_W // CHUNK
GSIZE = 100
NG = CHUNK // GSIZE


def _rsqrt16(v):
    i = lax.bitcast_convert_type(v, jnp.int32)
    y = lax.bitcast_convert_type(
        jnp.int32(0x5F3759DF) - lax.shift_right_arithmetic(i, 1), jnp.float32)
    for _ in range(3):
        y = y * (1.5 - 0.5 * v * y * y)
    return y


def _sc_body(ids_hbm, tok_hbm, pos_hbm, gb_hbm, out_hbm,
             idx_v, rows_v, pos_v, gb_v, sem):
    wid = lax.axis_index("s") * NC + lax.axis_index("c")

    pltpu.sync_copy(pos_hbm.at[pl.ds(0, L)], pos_v)
    pltpu.sync_copy(gb_hbm, gb_v)
    g = [gb_v[0, pl.ds(16 * j, 16)] for j in range(4)]
    bt = [gb_v[1, pl.ds(16 * j, 16)] for j in range(4)]

    def chunk_body(c, carry):
        base = wid * ROWS_PER_W + c * CHUNK
        idx_row = wid * (ROWS_PER_W // GSIZE) + c * NG
        pltpu.sync_copy(ids_hbm.at[pl.ds(idx_row, NG)], idx_v)
        cps = [pltpu.async_copy(tok_hbm.at[idx_v.at[j]],
                                rows_v.at[pl.ds(j * GSIZE, GSIZE)], sem)
               for j in range(NG)]
        for cp in cps:
            cp.wait()

        def period_body(p, carry2):
            def row_body(l, carry3):
                r = p * L + l
                x = [rows_v[r, pl.ds(16 * j, 16)] + pos_v[l, pl.ds(16 * j, 16)]
                     for j in range(4)]
                s = (x[0] + x[1]) + (x[2] + x[3])
                mean = jnp.sum(s) * (1.0 / 64.0)
                q = (x[0] * x[0] + x[1] * x[1]) + (x[2] * x[2] + x[3] * x[3])
                var = jnp.sum(q) * (1.0 / 64.0) - mean * mean
                rstd = _rsqrt16(jnp.full((16,), var + EPS, jnp.float32))
                for j in range(4):
                    rows_v[r, pl.ds(16 * j, 16)] = (
                        (x[j] - mean) * rstd * g[j] + bt[j])
                return carry3
            return lax.fori_loop(0, L, row_body, carry2)
        lax.fori_loop(0, CHUNK // L, period_body, 0)

        pltpu.sync_copy(rows_v, out_hbm.at[pl.ds(base, CHUNK)])
        return carry
    lax.fori_loop(0, NCHUNK, chunk_body, 0)


@functools.partial(jax.jit, static_argnums=())
def _sc_call(ids, token_table, pos_table, gb):
    mesh = plsc.VectorSubcoreMesh(core_axis_name="c", subcore_axis_name="s")
    f = functools.partial(
        pl.kernel,
        mesh=mesh,
        out_type=jax.ShapeDtypeStruct((BL, EMBED), jnp.float32),
        compiler_params=pltpu.CompilerParams(
            needs_layout_passes=False, use_tc_tiling_on_sc=False),
        scratch_types=[
            pltpu.VMEM((NG, GSIZE), jnp.int32),
            pltpu.VMEM((CHUNK, EMBED), jnp.float32),
            pltpu.VMEM((L, EMBED), jnp.float32),
            pltpu.VMEM((2, EMBED), jnp.float32),
            pltpu.SemaphoreType.DMA,
        ],
    )(_sc_body)
    return f(ids, token_table, pos_table, gb)


def kernel(input_ids, token_table, pos_table, ln_gamma, ln_beta):
    ids = input_ids.reshape(BL // GSIZE, GSIZE).astype(jnp.int32)
    gb = jnp.stack([ln_gamma, ln_beta])
    out = _sc_call(ids, token_table, pos_table, gb)
    return out.reshape(B, L, EMBED)
